# baseline (speedup 1.0000x reference)
import jax
import jax.numpy as jnp
from jax import lax
from jax.experimental import pallas as pl
from jax.experimental.pallas import tpu as pltpu
from jax.experimental.pallas import tpu_sc as plsc

_HW = 112 * 112
_K = int(0.5 * _HW)
_PLANES = 4 * 192
_NW = 32
_PPW = _PLANES // _NW
_NV = _HW // 16
_NB = 1 << 11
_CCAP = 64


def _sc_body(x_hbm, o_hbm, bin0, bin1, bout0, bout1, hist, sfx, cand,
             isem0, isem1, osem0, osem1):
    wid = lax.axis_index("s") * 2 + lax.axis_index("c")
    base = wid * _PPW
    lane_iota = lax.iota(jnp.int32, 16)
    zeros16 = jnp.zeros((16,), jnp.int32)
    bins = (bin0, bin1)
    bouts = (bout0, bout1)
    isems = (isem0, isem1)
    osems = (osem0, osem1)

    pltpu.async_copy(x_hbm.at[base], bin0, isem0)
    pltpu.async_copy(x_hbm.at[base + 1], bin1, isem1)

    @plsc.parallel_loop(0, _NB // 16, unroll=8)
    def zero_it(jj):
        hist[pl.ds(jj * 16, 16)] = zeros16

    def scan_hist(width, k_rem):
        nchunks = (1 << width) // 16
        ngroups = nchunks // 16

        @plsc.parallel_loop(0, nchunks, unroll=4)
        def scanA(c):
            t = hist[pl.ds(c * 16, 16)]
            hist[pl.ds(c * 16, 16)] = zeros16
            sfx[pl.ds(c * 16, 16)] = plsc.cumsum(lax.rev(t, (0,)))

        def scanB(gg, carry):
            csfx, cnt_v, snext_v = carry
            g = ngroups - 1 - gg
            idx = g * 256 + lane_iota * 16 + 15
            tot16 = plsc.load_gather(sfx, [idx])
            sfx_incl = lax.rev(plsc.cumsum(lax.rev(tot16, (0,))), (0,))
            rvec = sfx_incl - tot16 + csfx
            for jc in range(16):
                c = g * 16 + jc
                u = sfx[pl.ds(c * 16, 16)]
                rj = jnp.take_along_axis(
                    rvec, jnp.full((16,), jc, jnp.int32), axis=0)
                s = lax.rev(u, (0,)) + rj
                cnt_v = cnt_v + jnp.where(s >= k_rem, 1, 0)
                snext_v = jnp.maximum(snext_v, jnp.where(s < k_rem, s, 0))
            gtot = jnp.take_along_axis(
                sfx_incl, jnp.full((16,), 0, jnp.int32), axis=0)
            return csfx + gtot, cnt_v, snext_v

        _, cnt_v, snext_v = lax.fori_loop(
            0, ngroups, scanB, (zeros16, zeros16, zeros16))
        return jnp.sum(cnt_v) - 1, jnp.max(snext_v)

    def process(plane, buf, bout, isem, osem, j):
        pltpu.make_async_copy(x_hbm.at[plane], buf, isem).wait()

        @plsc.parallel_loop(0, _NV, unroll=8)
        def hist_it(i):
            v = buf[pl.ds(i * 16, 16)]
            fld = (lax.bitcast_convert_type(v, jnp.int32)
                   & 0x7FFFFFFF) >> 20
            cnts, last = plsc.scan_count(fld)
            plsc.addupdate_scatter(hist, [fld], cnts, mask=last)

        b1, snext = scan_hist(11, jnp.int32(_K))
        k_rem = jnp.int32(_K) - snext

        @plsc.parallel_loop(0, _NV, unroll=8, carry=zeros16)
        def comp_it(i, off_v):
            ab = (lax.bitcast_convert_type(buf[pl.ds(i * 16, 16)],
                                           jnp.int32) & 0x7FFFFFFF)
            m = (ab >> 20) == b1
            mi = jnp.where(m, 1, 0)
            pos = plsc.cumsum(mi) - mi + off_v
            plsc.store_scatter(cand, [pos], ab & 0xFFFFF, mask=m)
            return off_v + plsc.all_reduce_population_count(m)

        n_cand = jnp.max(comp_it)

        def small_tail(_):
            vs = [cand[pl.ds(i * 16, 16)] for i in range(_CCAP // 16)]
            valids = [(lane_iota + i * 16) < n_cand
                      for i in range(_CCAP // 16)]

            def bs_it(_, carry):
                lo, hi = carry
                mid = lo + ((hi - lo + 1) >> 1)
                acc = zeros16
                for v, valid in zip(vs, valids):
                    acc = acc + jnp.where(valid & (v >= mid), 1, 0)
                ok = jnp.sum(acc) >= k_rem
                return jnp.where(ok, mid, lo), jnp.where(ok, hi, mid - 1)

            lo, _ = lax.fori_loop(0, 20, bs_it,
                                  (jnp.int32(0), jnp.int32(0xFFFFF)))
            return (b1 << 20) | lo

        def big_tail(_):
            prefix = b1
            kr = k_rem
            for shift, width in ((10, 10), (0, 10)):
                @plsc.parallel_loop(0, _NV, unroll=8)
                def hist_it(i, shift=shift, width=width, prefix=prefix):
                    v = buf[pl.ds(i * 16, 16)]
                    b = (lax.bitcast_convert_type(v, jnp.int32)
                         & 0x7FFFFFFF) >> shift
                    sel = (b >> width) == prefix
                    fld = b & ((1 << width) - 1)
                    cnts, last = plsc.scan_count(fld, mask=sel)
                    plsc.addupdate_scatter(hist, [fld], cnts, mask=last)

                bstar, snext = scan_hist(width, kr)
                prefix = (prefix << width) | bstar
                kr = kr - snext
            return prefix

        thr = lax.cond(n_cand <= _CCAP, small_tail, big_tail, 0)

        @pl.when(j >= 2)
        def _():
            pltpu.make_async_copy(bout, o_hbm.at[plane], osem).wait()

        @plsc.parallel_loop(0, _NV, unroll=8)
        def mask_it(i):
            v = buf[pl.ds(i * 16, 16)]
            ab = lax.bitcast_convert_type(v, jnp.int32) & 0x7FFFFFFF
            bout[pl.ds(i * 16, 16)] = jnp.where(ab >= thr, v, 0.0)

        pltpu.async_copy(bout, o_hbm.at[plane], osem)

        @pl.when(j + 2 < _PPW)
        def _():
            pltpu.async_copy(x_hbm.at[plane + 2], buf, isem)

    @pl.loop(0, _PPW, step=2)
    def plane_loop(j):
        for b in range(2):
            process(base + j + b, bins[b], bouts[b], isems[b], osems[b],
                    j + b)

    pltpu.make_async_copy(bout0, o_hbm.at[base + _PPW - 2], osem0).wait()
    pltpu.make_async_copy(bout1, o_hbm.at[base + _PPW - 1], osem1).wait()


@jax.jit
def _sc_call(x2):
    return pl.kernel(
        _sc_body,
        out_type=jax.ShapeDtypeStruct((_PLANES, _HW), jnp.float32),
        mesh=plsc.VectorSubcoreMesh(core_axis_name="c", subcore_axis_name="s"),
        compiler_params=pltpu.CompilerParams(needs_layout_passes=False),
        scratch_types=[
            pltpu.VMEM((_HW,), jnp.float32),
            pltpu.VMEM((_HW,), jnp.float32),
            pltpu.VMEM((_HW,), jnp.float32),
            pltpu.VMEM((_HW,), jnp.float32),
            pltpu.VMEM((_NB,), jnp.int32),
            pltpu.VMEM((_NB,), jnp.int32),
            pltpu.VMEM((_HW + 16,), jnp.int32),
            pltpu.SemaphoreType.DMA,
            pltpu.SemaphoreType.DMA,
            pltpu.SemaphoreType.DMA,
            pltpu.SemaphoreType.DMA,
        ],
    )(x2)


def kernel(x):
    B, C, H, W = x.shape
    x2 = x.reshape(B * C, H * W)
    return _sc_call(x2).reshape(B, C, H, W)


# speedup vs baseline: 1.1787x; 1.1787x over previous
import jax
import jax.numpy as jnp
from jax import lax
from jax.experimental import pallas as pl
from jax.experimental.pallas import tpu as pltpu
from jax.experimental.pallas import tpu_sc as plsc

_HW = 112 * 112
_K = int(0.5 * _HW)
_PLANES = 4 * 192
_NW = 32
_PPW = _PLANES // _NW
_NV = _HW // 16
_PASSES = ((20, 11), (10, 10), (0, 10))
_NB = 1 << 11


def _sc_body(x_hbm, o_hbm, bin0, bin1, bout0, bout1, hist, sfx,
             isem0, isem1, osem0, osem1):
    wid = lax.axis_index("s") * 2 + lax.axis_index("c")
    base = wid * _PPW
    lane_iota = lax.iota(jnp.int32, 16)
    zeros16 = jnp.zeros((16,), jnp.int32)
    bins = (bin0, bin1)
    bouts = (bout0, bout1)
    isems = (isem0, isem1)
    osems = (osem0, osem1)

    pltpu.async_copy(x_hbm.at[base], bin0, isem0)
    pltpu.async_copy(x_hbm.at[base + 1], bin1, isem1)

    @plsc.parallel_loop(0, _NB // 16, unroll=8)
    def zero_it(jj):
        hist[pl.ds(jj * 16, 16)] = zeros16

    def process(plane, buf, bout, isem, osem, j):
        pltpu.make_async_copy(x_hbm.at[plane], buf, isem).wait()

        prefix = jnp.int32(0)
        k_rem = jnp.int32(_K)
        for pidx, (shift, width) in enumerate(_PASSES):
            nb = 1 << width
            nchunks = nb // 16
            ngroups = nchunks // 16

            if pidx == 0:
                @plsc.parallel_loop(0, _NV, unroll=8)
                def hist_it(i, shift=shift, width=width):
                    v = buf[pl.ds(i * 16, 16)]
                    fld = (lax.bitcast_convert_type(v, jnp.int32)
                           & 0x7FFFFFFF) >> shift
                    cnts, last = plsc.scan_count(fld)
                    plsc.addupdate_scatter(hist, [fld], cnts, mask=last)
            else:
                @plsc.parallel_loop(0, _NV, unroll=8)
                def hist_it(i, shift=shift, width=width, prefix=prefix):
                    v = buf[pl.ds(i * 16, 16)]
                    b = (lax.bitcast_convert_type(v, jnp.int32)
                         & 0x7FFFFFFF) >> shift
                    sel = (b >> width) == prefix
                    fld = b & (nb - 1)
                    cnts, last = plsc.scan_count(fld, mask=sel)
                    plsc.addupdate_scatter(hist, [fld], cnts, mask=last)

            @plsc.parallel_loop(0, nchunks, unroll=4)
            def scanA(c):
                t = hist[pl.ds(c * 16, 16)]
                hist[pl.ds(c * 16, 16)] = zeros16
                sfx[pl.ds(c * 16, 16)] = plsc.cumsum(lax.rev(t, (0,)))

            def scanB(gg, carry):
                csfx, cnt_v, snext_v = carry
                g = ngroups - 1 - gg
                idx = g * 256 + lane_iota * 16 + 15
                tot16 = plsc.load_gather(sfx, [idx])
                sfx_incl = lax.rev(plsc.cumsum(lax.rev(tot16, (0,))), (0,))
                rvec = sfx_incl - tot16 + csfx
                for jc in range(16):
                    c = g * 16 + jc
                    u = sfx[pl.ds(c * 16, 16)]
                    rj = jnp.take_along_axis(
                        rvec, jnp.full((16,), jc, jnp.int32), axis=0)
                    s = lax.rev(u, (0,)) + rj
                    cnt_v = cnt_v + jnp.where(s >= k_rem, 1, 0)
                    snext_v = jnp.maximum(snext_v,
                                          jnp.where(s < k_rem, s, 0))
                gtot = jnp.take_along_axis(
                    sfx_incl, jnp.full((16,), 0, jnp.int32), axis=0)
                return csfx + gtot, cnt_v, snext_v

            _, cnt_v, snext_v = lax.fori_loop(
                0, ngroups, scanB, (zeros16, zeros16, zeros16))
            bstar = jnp.sum(cnt_v) - 1
            prefix = (prefix << width) | bstar
            k_rem = k_rem - jnp.max(snext_v)

        thr = prefix

        @pl.when(j >= 2)
        def _():
            pltpu.make_async_copy(bout, o_hbm.at[plane], osem).wait()

        @plsc.parallel_loop(0, _NV, unroll=8)
        def mask_it(i):
            v = buf[pl.ds(i * 16, 16)]
            ab = lax.bitcast_convert_type(v, jnp.int32) & 0x7FFFFFFF
            bout[pl.ds(i * 16, 16)] = jnp.where(ab >= thr, v, 0.0)

        pltpu.async_copy(bout, o_hbm.at[plane], osem)

        @pl.when(j + 2 < _PPW)
        def _():
            pltpu.async_copy(x_hbm.at[plane + 2], buf, isem)

    @pl.loop(0, _PPW, step=2)
    def plane_loop(j):
        for b in range(2):
            process(base + j + b, bins[b], bouts[b], isems[b], osems[b],
                    j + b)

    pltpu.make_async_copy(bout0, o_hbm.at[base + _PPW - 2], osem0).wait()
    pltpu.make_async_copy(bout1, o_hbm.at[base + _PPW - 1], osem1).wait()


@jax.jit
def _sc_call(x2):
    return pl.kernel(
        _sc_body,
        out_type=jax.ShapeDtypeStruct((_PLANES, _HW), jnp.float32),
        mesh=plsc.VectorSubcoreMesh(core_axis_name="c", subcore_axis_name="s"),
        compiler_params=pltpu.CompilerParams(needs_layout_passes=False),
        scratch_types=[
            pltpu.VMEM((_HW,), jnp.float32),
            pltpu.VMEM((_HW,), jnp.float32),
            pltpu.VMEM((_HW,), jnp.float32),
            pltpu.VMEM((_HW,), jnp.float32),
            pltpu.VMEM((_NB,), jnp.int32),
            pltpu.VMEM((_NB,), jnp.int32),
            pltpu.SemaphoreType.DMA,
            pltpu.SemaphoreType.DMA,
            pltpu.SemaphoreType.DMA,
            pltpu.SemaphoreType.DMA,
        ],
    )(x2)


def kernel(x):
    B, C, H, W = x.shape
    x2 = x.reshape(B * C, H * W)
    return _sc_call(x2).reshape(B, C, H, W)
